# Initial kernel scaffold; baseline (speedup 1.0000x reference)
import jax
import jax.numpy as jnp
from jax.experimental import pallas as pl


def kernel(x_syn, W1, b1, g1, beta1, W2, b2, g2, beta2, W3, b3, rows, cols, batch):
    raise NotImplementedError("write your pallas kernel here")


# speedup vs baseline: 1.3908x; 1.3908x over previous
import functools

import jax
import jax.numpy as jnp
from jax import lax
from jax.experimental import pallas as pl
from jax.experimental.pallas import tpu as pltpu
from jax.experimental.pallas import tpu_sc as plsc

N = 8192
XC = 128
H = 256
E = 65536
NN = N * N
LOG2N = 13

NC = 2
NS = 16
NW = NC * NS
L = 16
EPW = E // NW
CHUNK = 128
NCH = EPW // CHUNK

BE = 2048
GRID = E // BE

_f32 = jnp.float32
_i32 = jnp.int32


def _mesh():
    return plsc.VectorSubcoreMesh(
        core_axis_name="c", subcore_axis_name="s", num_cores=NC, num_subcores=NS
    )


def _wid():
    return lax.axis_index("s") * NC + lax.axis_index("c")


def _sc_gather(x_syn, rows, cols):
    @functools.partial(
        pl.kernel,
        out_type=(
            jax.ShapeDtypeStruct((E, XC), _f32),
            jax.ShapeDtypeStruct((E, XC), _f32),
        ),
        mesh=_mesh(),
        scratch_types=[
            pltpu.VMEM((CHUNK,), _i32),
            pltpu.VMEM((CHUNK, XC), _f32),
            pltpu.SemaphoreType.DMA,
        ],
    )
    def k(x_hbm, rows_hbm, cols_hbm, out_r, out_c, idx_v, buf, sem):
        base = _wid() * EPW

        @pl.loop(0, NCH)
        def _(t):
            off = base + t * CHUNK
            pltpu.sync_copy(rows_hbm.at[pl.ds(off, CHUNK)], idx_v)
            pltpu.async_copy(x_hbm.at[idx_v], buf, sem).wait()
            pltpu.sync_copy(buf, out_r.at[pl.ds(off, CHUNK)])
            pltpu.sync_copy(cols_hbm.at[pl.ds(off, CHUNK)], idx_v)
            pltpu.async_copy(x_hbm.at[idx_v], buf, sem).wait()
            pltpu.sync_copy(buf, out_c.at[pl.ds(off, CHUNK)])

    return k(x_syn, rows, cols)


def _mlp1(f_r, f_c, W1, b1):
    w1r = W1[:XC]
    w1c = W1[XC:]
    b1r = b1.reshape(1, H)

    def body(fr_ref, fc_ref, wr_ref, wc_ref, b_ref, h_ref, st_ref, acc_s, acc_q):
        h = jnp.dot(fr_ref[...], wr_ref[...], preferred_element_type=_f32)
        h = h + jnp.dot(fc_ref[...], wc_ref[...], preferred_element_type=_f32)
        h = h + b_ref[...]
        h_ref[...] = h
        i = pl.program_id(0)

        @pl.when(i == 0)
        def _():
            acc_s[...] = jnp.zeros_like(acc_s)
            acc_q[...] = jnp.zeros_like(acc_q)

        acc_s[...] += jnp.sum(h, axis=0, keepdims=True)
        acc_q[...] += jnp.sum(h * h, axis=0, keepdims=True)

        @pl.when(i == GRID - 1)
        def _():
            st_ref[0:1] = acc_s[...]
            st_ref[1:2] = acc_q[...]

    return pl.pallas_call(
        body,
        grid=(GRID,),
        in_specs=[
            pl.BlockSpec((BE, XC), lambda i: (i, 0)),
            pl.BlockSpec((BE, XC), lambda i: (i, 0)),
            pl.BlockSpec((XC, H), lambda i: (0, 0)),
            pl.BlockSpec((XC, H), lambda i: (0, 0)),
            pl.BlockSpec((1, H), lambda i: (0, 0)),
        ],
        out_specs=[
            pl.BlockSpec((BE, H), lambda i: (i, 0)),
            pl.BlockSpec((2, H), lambda i: (0, 0)),
        ],
        out_shape=[
            jax.ShapeDtypeStruct((E, H), _f32),
            jax.ShapeDtypeStruct((2, H), _f32),
        ],
        scratch_shapes=[pltpu.VMEM((1, H), _f32), pltpu.VMEM((1, H), _f32)],
    )(f_r, f_c, w1r, w1c, b1r)


def _bn_scale_shift(st_ref, g_ref, bt_ref):
    s = st_ref[0:1]
    q = st_ref[1:2]
    mean = s * (1.0 / E)
    var = q * (1.0 / E) - mean * mean
    scale = g_ref[...] * lax.rsqrt(var + 1e-5)
    shift = bt_ref[...] - mean * scale
    return scale, shift


def _mlp2(h1, st1, g1, beta1, W2, b2):
    g1r = g1.reshape(1, H)
    bt1r = beta1.reshape(1, H)
    b2r = b2.reshape(1, H)

    def body(h1_ref, st_ref, g_ref, bt_ref, w_ref, b_ref, h_ref, st2_ref, acc_s, acc_q):
        scale, shift = _bn_scale_shift(st_ref, g_ref, bt_ref)
        a = jnp.maximum(h1_ref[...] * scale + shift, 0.0)
        h = jnp.dot(a, w_ref[...], preferred_element_type=_f32) + b_ref[...]
        h_ref[...] = h
        i = pl.program_id(0)

        @pl.when(i == 0)
        def _():
            acc_s[...] = jnp.zeros_like(acc_s)
            acc_q[...] = jnp.zeros_like(acc_q)

        acc_s[...] += jnp.sum(h, axis=0, keepdims=True)
        acc_q[...] += jnp.sum(h * h, axis=0, keepdims=True)

        @pl.when(i == GRID - 1)
        def _():
            st2_ref[0:1] = acc_s[...]
            st2_ref[1:2] = acc_q[...]

    return pl.pallas_call(
        body,
        grid=(GRID,),
        in_specs=[
            pl.BlockSpec((BE, H), lambda i: (i, 0)),
            pl.BlockSpec((2, H), lambda i: (0, 0)),
            pl.BlockSpec((1, H), lambda i: (0, 0)),
            pl.BlockSpec((1, H), lambda i: (0, 0)),
            pl.BlockSpec((H, H), lambda i: (0, 0)),
            pl.BlockSpec((1, H), lambda i: (0, 0)),
        ],
        out_specs=[
            pl.BlockSpec((BE, H), lambda i: (i, 0)),
            pl.BlockSpec((2, H), lambda i: (0, 0)),
        ],
        out_shape=[
            jax.ShapeDtypeStruct((E, H), _f32),
            jax.ShapeDtypeStruct((2, H), _f32),
        ],
        scratch_shapes=[pltpu.VMEM((1, H), _f32), pltpu.VMEM((1, H), _f32)],
    )(h1, st1, g1r, bt1r, W2, b2r)


def _mlp3(h2, st2, g2, beta2, W3, b3):
    g2r = g2.reshape(1, H)
    bt2r = beta2.reshape(1, H)
    w3r = W3.reshape(1, H)

    def body(h2_ref, st_ref, g_ref, bt_ref, w_ref, b_ref, out_ref):
        scale, shift = _bn_scale_shift(st_ref, g_ref, bt_ref)
        a = jnp.maximum(h2_ref[...] * scale + shift, 0.0)
        logits = jnp.sum(a * w_ref[...], axis=1) + b_ref[...]
        out_ref[...] = jax.nn.sigmoid(logits)

    return pl.pallas_call(
        body,
        grid=(GRID,),
        in_specs=[
            pl.BlockSpec((BE, H), lambda i: (i, 0)),
            pl.BlockSpec((2, H), lambda i: (0, 0)),
            pl.BlockSpec((1, H), lambda i: (0, 0)),
            pl.BlockSpec((1, H), lambda i: (0, 0)),
            pl.BlockSpec((1, H), lambda i: (0, 0)),
            pl.BlockSpec((1,), lambda i: (0,)),
        ],
        out_specs=pl.BlockSpec((BE,), lambda i: (i,)),
        out_shape=jax.ShapeDtypeStruct((E,), _f32),
    )(h2, st2, g2r, bt2r, w3r, b3)


def _sc_scatter_ids(rows, cols):
    @functools.partial(
        pl.kernel,
        out_type=jax.ShapeDtypeStruct((NN,), _i32),
        mesh=_mesh(),
        scratch_types=[
            pltpu.VMEM((EPW,), _i32),
            pltpu.VMEM((EPW,), _i32),
            pltpu.VMEM((NCH, CHUNK), _i32),
            pltpu.VMEM((NCH, CHUNK), _i32),
            pltpu.SemaphoreType.DMA,
        ],
    )
    def k(rows_hbm, cols_hbm, m_out, rbuf, cbuf, kidx, ebuf, sem):
        base = _wid() * EPW
        pltpu.sync_copy(rows_hbm.at[pl.ds(base, EPW)], rbuf)
        pltpu.sync_copy(cols_hbm.at[pl.ds(base, EPW)], cbuf)

        @pl.loop(0, NCH)
        def _(j):
            @pl.loop(0, CHUNK // L)
            def _(q):
                o = j * CHUNK + q * L
                r = rbuf[pl.ds(o, L)]
                c = cbuf[pl.ds(o, L)]
                kidx[j, pl.ds(q * L, L)] = (r << LOG2N) | c
                ebuf[j, pl.ds(q * L, L)] = (
                    base + o + lax.iota(_i32, L)
                )

        @pl.loop(0, NCH)
        def _(j):
            pltpu.async_copy(ebuf.at[j], m_out.at[kidx.at[j]], sem).wait()

    return k(rows, cols)


def _sc_mask_sums(m, rows, cols, vals):
    @functools.partial(
        pl.kernel,
        out_type=(
            jax.ShapeDtypeStruct((4, N), _f32),
            jax.ShapeDtypeStruct((E,), _f32),
        ),
        mesh=_mesh(),
        scratch_types=[
            pltpu.VMEM((EPW,), _i32),
            pltpu.VMEM((EPW,), _i32),
            pltpu.VMEM((EPW,), _f32),
            pltpu.VMEM((NCH, CHUNK), _i32),
            pltpu.VMEM((NCH, CHUNK), _i32),
            pltpu.VMEM((EPW,), _i32),
            pltpu.VMEM((EPW,), _i32),
            pltpu.VMEM((EPW,), _i32),
            pltpu.VMEM((EPW,), _i32),
            pltpu.VMEM((EPW,), _f32),
            pltpu.VMEM((EPW,), _f32),
            pltpu.VMEM((NCH, CHUNK), _f32),
            pltpu.VMEM((NCH, CHUNK), _i32),
            pltpu.VMEM((NCH, CHUNK), _i32),
            pltpu.VMEM((EPW,), _f32),
            pltpu.VMEM_SHARED((N,), _f32),
            pltpu.VMEM_SHARED((N,), _f32),
            pltpu.SemaphoreType.DMA,
        ],
    )
    def k(m_hbm, rows_hbm, cols_hbm, vals_hbm, sums_out, ev_out,
          rbuf, cbuf, vbuf, kidx, krev, wbuf, wcbuf, rgbuf, cgbuf, vgbuf,
          evbuf, mv2d, r2d, c2d, zbuf, acc_r, acc_c, sem):
        sid = lax.axis_index("s")
        cid = lax.axis_index("c")
        base = _wid() * EPW

        @pl.when(sid == 0)
        def _():
            @pl.loop(0, EPW // L)
            def _(i):
                zbuf[pl.ds(i * L, L)] = jnp.zeros((L,), _f32)

            @pl.loop(0, N // EPW)
            def _(i):
                pltpu.sync_copy(zbuf, acc_r.at[pl.ds(i * EPW, EPW)])
                pltpu.sync_copy(zbuf, acc_c.at[pl.ds(i * EPW, EPW)])

        plsc.subcore_barrier()

        pltpu.sync_copy(rows_hbm.at[pl.ds(base, EPW)], rbuf)
        pltpu.sync_copy(cols_hbm.at[pl.ds(base, EPW)], cbuf)
        pltpu.sync_copy(vals_hbm.at[pl.ds(base, EPW)], vbuf)

        @pl.loop(0, NCH)
        def _(j):
            @pl.loop(0, CHUNK // L)
            def _(q):
                o = j * CHUNK + q * L
                r = rbuf[pl.ds(o, L)]
                c = cbuf[pl.ds(o, L)]
                kidx[j, pl.ds(q * L, L)] = (r << LOG2N) | c
                krev[j, pl.ds(q * L, L)] = (c << LOG2N) | r
                r2d[j, pl.ds(q * L, L)] = r
                c2d[j, pl.ds(q * L, L)] = c

        @pl.loop(0, NCH)
        def _(j):
            sl = pl.ds(j * CHUNK, CHUNK)
            pltpu.async_copy(m_hbm.at[kidx.at[j]], wbuf.at[sl], sem).wait()
            pltpu.async_copy(m_hbm.at[krev.at[j]], wcbuf.at[sl], sem).wait()

        @pl.loop(0, EPW // L)
        def _(i):
            sl = pl.ds(i * L, L)
            w = wcbuf[sl]
            wcbuf[sl] = jnp.minimum(jnp.maximum(w, 0), E - 1)

        @pl.loop(0, NCH)
        def _(j):
            sl = pl.ds(j * CHUNK, CHUNK)
            idx = wcbuf.at[sl]
            pltpu.async_copy(rows_hbm.at[idx], rgbuf.at[sl], sem).wait()
            pltpu.async_copy(cols_hbm.at[idx], cgbuf.at[sl], sem).wait()
            pltpu.async_copy(vals_hbm.at[idx], vgbuf.at[sl], sem).wait()

        @pl.loop(0, NCH)
        def _(j):
            @pl.loop(0, CHUNK // L)
            def _(q):
                o = j * CHUNK + q * L
                sl = pl.ds(o, L)
                r = rbuf[sl]
                c = cbuf[sl]
                v = vbuf[sl]
                e = base + o + lax.iota(_i32, L)
                mask = wbuf[sl] == e
                valid = (rgbuf[sl] == c) & (cgbuf[sl] == r)
                zero = jnp.zeros((L,), _f32)
                rev = jnp.where(valid, vgbuf[sl], zero)
                s = (v + rev) * 0.5
                one = jnp.full((L,), 1.0, _f32)
                evbuf[sl] = s + jnp.where(r == c, one, zero)
                mv2d[j, pl.ds(q * L, L)] = jnp.where(mask, v, zero)

        @pl.loop(0, NCH)
        def _(j):
            pltpu.sync_copy(mv2d.at[j], acc_r.at[r2d.at[j]], add=True)
            pltpu.sync_copy(mv2d.at[j], acc_c.at[c2d.at[j]], add=True)

        pltpu.sync_copy(evbuf, ev_out.at[pl.ds(base, EPW)])

        plsc.subcore_barrier()

        @pl.when(sid == 0)
        def _():
            pltpu.sync_copy(acc_r, sums_out.at[cid])
            pltpu.sync_copy(acc_c, sums_out.at[2 + cid])

    return k(m, rows, cols, vals)


BR = 256


def _tc_diag(sums):
    def body(s_ref, dense_ref, dinv_ref):
        i = pl.program_id(0)
        s = s_ref[...]
        deg = 1.0 + 0.5 * jnp.sum(s, axis=0)
        dv = lax.rsqrt(deg)
        dinv_ref[...] = dv
        ri = lax.broadcasted_iota(_i32, (BR, N), 0) + i * BR
        ci = lax.broadcasted_iota(_i32, (BR, N), 1)
        dense_ref[...] = jnp.where(ri == ci, (dv * dv)[:, None], 0.0)

    return pl.pallas_call(
        body,
        grid=(N // BR,),
        in_specs=[pl.BlockSpec((4, BR), lambda i: (0, i))],
        out_specs=[
            pl.BlockSpec((BR, N), lambda i: (i, 0)),
            pl.BlockSpec((BR,), lambda i: (i,)),
        ],
        out_shape=[
            jax.ShapeDtypeStruct((N, N), _f32),
            jax.ShapeDtypeStruct((N,), _f32),
        ],
    )(sums)


def _sc_final_scatter(dense_ref, rows, cols, ev, dinv):
    @functools.partial(
        pl.kernel,
        out_type=(),
        mesh=_mesh(),
        scratch_types=[
            pltpu.VMEM((EPW,), _i32),
            pltpu.VMEM((EPW,), _i32),
            pltpu.VMEM((EPW,), _f32),
            pltpu.VMEM((EPW,), _f32),
            pltpu.VMEM((EPW,), _f32),
            pltpu.VMEM((NCH, CHUNK), _i32),
            pltpu.VMEM((NCH, CHUNK), _i32),
            pltpu.VMEM((NCH, CHUNK), _f32),
            pltpu.SemaphoreType.DMA,
        ],
    )
    def k(dense, rows_hbm, cols_hbm, ev_hbm, dinv_hbm,
          rbuf, cbuf, evbuf, drbuf, dcbuf, k2d, krev2d, v2d, sem):
        base = _wid() * EPW
        pltpu.sync_copy(rows_hbm.at[pl.ds(base, EPW)], rbuf)
        pltpu.sync_copy(cols_hbm.at[pl.ds(base, EPW)], cbuf)
        pltpu.sync_copy(ev_hbm.at[pl.ds(base, EPW)], evbuf)

        @pl.loop(0, NCH)
        def _(j):
            sl = pl.ds(j * CHUNK, CHUNK)
            pltpu.async_copy(dinv_hbm.at[rbuf.at[sl]], drbuf.at[sl], sem).wait()
            pltpu.async_copy(dinv_hbm.at[cbuf.at[sl]], dcbuf.at[sl], sem).wait()

        @pl.loop(0, NCH)
        def _(j):
            @pl.loop(0, CHUNK // L)
            def _(q):
                o = j * CHUNK + q * L
                sl = pl.ds(o, L)
                r = rbuf[sl]
                c = cbuf[sl]
                k2d[j, pl.ds(q * L, L)] = (r << LOG2N) | c
                krev2d[j, pl.ds(q * L, L)] = (c << LOG2N) | r
                v2d[j, pl.ds(q * L, L)] = evbuf[sl] * drbuf[sl] * dcbuf[sl]

        @pl.loop(0, NCH)
        def _(j):
            pltpu.async_copy(v2d.at[j], dense.at[k2d.at[j]], sem).wait()
            pltpu.async_copy(v2d.at[j], dense.at[krev2d.at[j]], sem).wait()

    k(dense_ref, rows, cols, ev, dinv)


def kernel(x_syn, W1, b1, g1, beta1, W2, b2, g2, beta2, W3, b3, rows, cols, batch):
    f_r, f_c = _sc_gather(x_syn, rows, cols)
    h1, st1 = _mlp1(f_r, f_c, W1, b1)
    h2, st2 = _mlp2(h1, st1, g1, beta1, W2, b2)
    vals = _mlp3(h2, st2, g2, beta2, W3, b3)
    m = _sc_scatter_ids(rows, cols)
    sums, ev = _sc_mask_sums(m, rows, cols, vals)
    dense, dinv = _tc_diag(sums)
    dref = jax.new_ref(dense.reshape(NN))
    _sc_final_scatter(dref, rows, cols, ev, dinv)
    return jax.freeze(dref).reshape(1, N, N)
